# baseline (speedup 1.0000x reference)
import functools

import jax
import jax.numpy as jnp
from jax import lax
from jax.experimental import pallas as pl
from jax.experimental.pallas import tpu as pltpu, tpu_sc as plsc

_NE = 512
_D = 64
_BB = 8
_P = 256


def _tc_body(z_ref, e_ref, idx_ref):
    e_mat = e_ref[...]
    s = jnp.sum(e_mat * e_mat, axis=1, keepdims=True)
    jid = jax.lax.broadcasted_iota(jnp.int32, (_NE, _P), 0)
    for b in range(_BB):
        zb = z_ref[b]
        m = jax.lax.dot_general(
            e_mat, zb, (((1,), (0,)), ((), ())),
            preferred_element_type=jnp.float32,
            precision=jax.lax.Precision.DEFAULT)
        d = s - 2.0 * m
        dmin = jnp.min(d, axis=0, keepdims=True)
        cand = jnp.where(d == dmin, jid, jnp.int32(_NE))
        idx = jnp.min(cand, axis=0)
        idx_ref[b, :] = idx


def _tc_indices(z3, e):
    B = z3.shape[0]
    return pl.pallas_call(
        _tc_body,
        grid=(B // _BB,),
        in_specs=[
            pl.BlockSpec((_BB, _D, _P), lambda i: (i, 0, 0)),
            pl.BlockSpec((_NE, _D), lambda i: (0, 0)),
        ],
        out_specs=pl.BlockSpec((_BB, _P), lambda i: (i, 0)),
        out_shape=jax.ShapeDtypeStruct((B, _P), jnp.int32),
    )(z3, e)


def _sc_gather(e_t, idx, B):
    info = plsc.get_sparse_core_info()
    nc, ns = info.num_cores, info.num_subcores
    nw = nc * ns
    bpw = B // nw
    mesh = plsc.VectorSubcoreMesh(core_axis_name="c", subcore_axis_name="s")

    @functools.partial(
        pl.kernel,
        mesh=mesh,
        compiler_params=pltpu.CompilerParams(needs_layout_passes=False),
        out_type=jax.ShapeDtypeStruct((B, _D, _P), jnp.float32),
        scratch_types=[
            pltpu.VMEM((_D, _NE), jnp.float32),
            pltpu.VMEM((bpw, _P), jnp.int32),
            pltpu.VMEM((_D, _P), jnp.float32),
            pltpu.VMEM((_D, _P), jnp.float32),
            pltpu.SemaphoreType.DMA,
            pltpu.SemaphoreType.DMA,
        ],
    )
    def k(et_hbm, idx_hbm, out_hbm, et_v, idx_all, out0, out1, s0, s1):
        wid = lax.axis_index("s") * nc + lax.axis_index("c")
        base = wid * bpw
        pltpu.sync_copy(et_hbm, et_v)
        pltpu.sync_copy(idx_hbm.at[pl.ds(base, bpw)], idx_all)
        bufs, sems = (out0, out1), (s0, s1)
        cps = [None, None]
        for bi in range(bpw):
            buf, sem = bufs[bi % 2], sems[bi % 2]
            if cps[bi % 2] is not None:
                cps[bi % 2].wait()

            idx_vecs = [idx_all[bi, pl.ds(k * 16, 16)]
                        for k in range(_P // 16)]

            @plsc.parallel_loop(0, _D, unroll=4)
            def chan(ch):
                chv = jnp.full((16,), 0, jnp.int32) + ch
                for k in range(_P // 16):
                    vals = plsc.load_gather(et_v, [chv, idx_vecs[k]])
                    buf[ch, pl.ds(k * 16, 16)] = vals

            cps[bi % 2] = pltpu.async_copy(buf, out_hbm.at[base + bi], sem)
        for cp in cps:
            if cp is not None:
                cp.wait()

    return k(e_t, idx)


@functools.partial(jax.jit, static_argnums=())
def kernel(z_e, e):
    B, C, H, W = z_e.shape
    z3 = z_e.reshape(B, C, H * W)
    idx = _tc_indices(z3, e)
    zq = _sc_gather(e.T, idx, B)
    return zq.reshape(B, C, H, W)


# speedup vs baseline: 1.0262x; 1.0262x over previous
import functools

import jax
import jax.numpy as jnp
from jax import lax
from jax.experimental import pallas as pl
from jax.experimental.pallas import tpu as pltpu, tpu_sc as plsc

_NE = 512
_D = 64
_BB = 16
_P = 256


def _tc_body(z_ref, e_ref, idx_ref):
    e_mat = e_ref[...]
    s = jnp.sum(e_mat * e_mat, axis=1, keepdims=True)
    jid = jax.lax.broadcasted_iota(jnp.int32, (_NE, _P), 0)
    for b in range(_BB):
        zb = z_ref[b]
        m = jax.lax.dot_general(
            e_mat, zb, (((1,), (0,)), ((), ())),
            preferred_element_type=jnp.float32,
            precision=jax.lax.Precision.DEFAULT)
        d = s - 2.0 * m
        dmin = jnp.min(d, axis=0, keepdims=True)
        cand = jnp.where(d == dmin, jid, jnp.int32(_NE))
        idx = jnp.min(cand, axis=0)
        idx_ref[b, :] = idx


def _tc_indices(z3, e):
    B = z3.shape[0]
    return pl.pallas_call(
        _tc_body,
        grid=(B // _BB,),
        in_specs=[
            pl.BlockSpec((_BB, _D, _P), lambda i: (i, 0, 0)),
            pl.BlockSpec((_NE, _D), lambda i: (0, 0)),
        ],
        out_specs=pl.BlockSpec((_BB, _P), lambda i: (i, 0)),
        out_shape=jax.ShapeDtypeStruct((B, _P), jnp.int32),
    )(z3, e)


def _sc_gather(e_t, idx, B):
    info = plsc.get_sparse_core_info()
    nc, ns = info.num_cores, info.num_subcores
    nw = nc * ns
    bpw = B // nw
    mesh = plsc.VectorSubcoreMesh(core_axis_name="c", subcore_axis_name="s")

    @functools.partial(
        pl.kernel,
        mesh=mesh,
        compiler_params=pltpu.CompilerParams(needs_layout_passes=False),
        out_type=jax.ShapeDtypeStruct((B, _D, _P), jnp.float32),
        scratch_types=[
            pltpu.VMEM((_D, _NE), jnp.float32),
            pltpu.VMEM((bpw, _P), jnp.int32),
            pltpu.VMEM((_D, _P), jnp.float32),
            pltpu.VMEM((_D, _P), jnp.float32),
            pltpu.SemaphoreType.DMA,
            pltpu.SemaphoreType.DMA,
        ],
    )
    def k(et_hbm, idx_hbm, out_hbm, et_v, idx_all, out0, out1, s0, s1):
        wid = lax.axis_index("s") * nc + lax.axis_index("c")
        base = wid * bpw
        pltpu.sync_copy(et_hbm, et_v)
        pltpu.sync_copy(idx_hbm.at[pl.ds(base, bpw)], idx_all)
        bufs, sems = (out0, out1), (s0, s1)
        cps = [None, None]
        for bi in range(bpw):
            buf, sem = bufs[bi % 2], sems[bi % 2]
            if cps[bi % 2] is not None:
                cps[bi % 2].wait()

            idx_vecs = [idx_all[bi, pl.ds(k * 16, 16)]
                        for k in range(_P // 16)]

            @plsc.parallel_loop(0, _D, unroll=2)
            def chan(ch):
                chv = jnp.full((16,), 0, jnp.int32) + ch
                for k in range(_P // 16):
                    vals = plsc.load_gather(et_v, [chv, idx_vecs[k]])
                    buf[ch, pl.ds(k * 16, 16)] = vals

            cps[bi % 2] = pltpu.async_copy(buf, out_hbm.at[base + bi], sem)
        for cp in cps:
            if cp is not None:
                cp.wait()

    return k(e_t, idx)


@functools.partial(jax.jit, static_argnums=())
def kernel(z_e, e):
    B, C, H, W = z_e.shape
    z3 = z_e.reshape(B, C, H * W)
    idx = _tc_indices(z3, e)
    zq = _sc_gather(e.T, idx, B)
    return zq.reshape(B, C, H, W)


# speedup vs baseline: 1.1105x; 1.0821x over previous
import functools

import jax
import jax.numpy as jnp
from jax import lax
from jax.experimental import pallas as pl
from jax.experimental.pallas import tpu as pltpu, tpu_sc as plsc

_NE = 512
_D = 64
_BB = 16
_P = 256


def _tc_body(z_ref, e_ref, idx_ref):
    e_mat = e_ref[...]
    s = jnp.sum(e_mat * e_mat, axis=1, keepdims=True)
    jid = jax.lax.broadcasted_iota(jnp.int32, (_NE, _P), 0)
    for b in range(_BB):
        zb = z_ref[b]
        m = jax.lax.dot_general(
            e_mat, zb, (((1,), (0,)), ((), ())),
            preferred_element_type=jnp.float32,
            precision=jax.lax.Precision.DEFAULT)
        d = s - 2.0 * m
        idx = jnp.argmin(d, axis=0).astype(jnp.int32)
        idx_ref[b, :] = idx


def _tc_indices(z3, e):
    B = z3.shape[0]
    return pl.pallas_call(
        _tc_body,
        grid=(B // _BB,),
        in_specs=[
            pl.BlockSpec((_BB, _D, _P), lambda i: (i, 0, 0)),
            pl.BlockSpec((_NE, _D), lambda i: (0, 0)),
        ],
        out_specs=pl.BlockSpec((_BB, _P), lambda i: (i, 0)),
        out_shape=jax.ShapeDtypeStruct((B, _P), jnp.int32),
    )(z3, e)


def _sc_gather(e_t, idx, B):
    info = plsc.get_sparse_core_info()
    nc, ns = info.num_cores, info.num_subcores
    nw = nc * ns
    bpw = B // nw
    mesh = plsc.VectorSubcoreMesh(core_axis_name="c", subcore_axis_name="s")

    @functools.partial(
        pl.kernel,
        mesh=mesh,
        compiler_params=pltpu.CompilerParams(needs_layout_passes=False),
        out_type=jax.ShapeDtypeStruct((B, _D, _P), jnp.float32),
        scratch_types=[
            pltpu.VMEM((_D, _NE), jnp.float32),
            pltpu.VMEM((bpw, _P), jnp.int32),
            pltpu.VMEM((_D, _P), jnp.float32),
            pltpu.VMEM((_D, _P), jnp.float32),
            pltpu.SemaphoreType.DMA,
            pltpu.SemaphoreType.DMA,
        ],
    )
    def k(et_hbm, idx_hbm, out_hbm, et_v, idx_all, out0, out1, s0, s1):
        wid = lax.axis_index("s") * nc + lax.axis_index("c")
        base = wid * bpw
        pltpu.sync_copy(et_hbm, et_v)
        pltpu.sync_copy(idx_hbm.at[pl.ds(base, bpw)], idx_all)
        bufs, sems = (out0, out1), (s0, s1)
        cps = [None, None]
        for bi in range(bpw):
            buf, sem = bufs[bi % 2], sems[bi % 2]
            if cps[bi % 2] is not None:
                cps[bi % 2].wait()

            idx_vecs = [idx_all[bi, pl.ds(k * 16, 16)]
                        for k in range(_P // 16)]

            @plsc.parallel_loop(0, _D, unroll=2)
            def chan(ch):
                chv = jnp.full((16,), 0, jnp.int32) + ch
                for k in range(_P // 16):
                    vals = plsc.load_gather(et_v, [chv, idx_vecs[k]])
                    buf[ch, pl.ds(k * 16, 16)] = vals

            cps[bi % 2] = pltpu.async_copy(buf, out_hbm.at[base + bi], sem)
        for cp in cps:
            if cp is not None:
                cp.wait()

    return k(e_t, idx)


@functools.partial(jax.jit, static_argnums=())
def kernel(z_e, e):
    B, C, H, W = z_e.shape
    z3 = z_e.reshape(B, C, H * W)
    idx = _tc_indices(z3, e)
    zq = _sc_gather(e.T, idx, B)
    return zq.reshape(B, C, H, W)


# speedup vs baseline: 1.2565x; 1.1315x over previous
import functools

import jax
import jax.numpy as jnp
from jax import lax
from jax.experimental import pallas as pl
from jax.experimental.pallas import tpu as pltpu, tpu_sc as plsc

_NE = 512
_D = 64
_BB = 16
_P = 256


def _tc_body(z_ref, e_ref, idx_ref):
    e_mat = e_ref[...]
    s = jnp.sum(e_mat * e_mat, axis=1, keepdims=True)
    jid = jax.lax.broadcasted_iota(jnp.int32, (_NE, _P), 0)
    for b in range(_BB):
        zb = z_ref[b]
        m = jax.lax.dot_general(
            e_mat, zb, (((1,), (0,)), ((), ())),
            preferred_element_type=jnp.float32,
            precision=jax.lax.Precision.DEFAULT)
        d = s - 2.0 * m
        idx = jnp.argmin(d, axis=0).astype(jnp.int32)
        idx_ref[b, :] = idx


def _tc_indices(z3, e):
    B = z3.shape[0]
    return pl.pallas_call(
        _tc_body,
        grid=(B // _BB,),
        in_specs=[
            pl.BlockSpec((_BB, _D, _P), lambda i: (i, 0, 0)),
            pl.BlockSpec((_NE, _D), lambda i: (0, 0)),
        ],
        out_specs=pl.BlockSpec((_BB, _P), lambda i: (i, 0)),
        out_shape=jax.ShapeDtypeStruct((B, _P), jnp.int32),
    )(z3, e)


def _sc_gather(e_t_flat, idx, B):
    info = plsc.get_sparse_core_info()
    nc, ns = info.num_cores, info.num_subcores
    nw = nc * ns
    ppw = _P // nw
    mesh = plsc.VectorSubcoreMesh(core_axis_name="c", subcore_axis_name="s")

    @functools.partial(
        pl.kernel,
        mesh=mesh,
        compiler_params=pltpu.CompilerParams(needs_layout_passes=False),
        out_type=jax.ShapeDtypeStruct((_D, _P, B), jnp.float32),
        scratch_types=[
            pltpu.VMEM((_NE * _D,), jnp.float32),
            pltpu.VMEM((ppw, B), jnp.int32),
            pltpu.VMEM((_D, ppw, B), jnp.float32),
        ],
    )
    def k(etf_hbm, idx_hbm, out_hbm, etf_v, idx_v, buf):
        wid = lax.axis_index("s") * nc + lax.axis_index("c")
        p0 = wid * ppw
        pltpu.sync_copy(etf_hbm, etf_v)
        pltpu.sync_copy(idx_hbm.at[pl.ds(p0, ppw)], idx_v)

        @plsc.parallel_loop(0, ppw * (B // 16), unroll=2)
        def pair(i):
            p = i // (B // 16)
            bc = i % (B // 16)
            idx16 = idx_v[p, pl.ds(bc * 16, 16)]
            for c in range(_D):
                vals = plsc.load_gather(etf_v, [idx16 + c * _NE])
                buf[c, i // (B // 16), pl.ds(bc * 16, 16)] = vals

        pltpu.sync_copy(buf, out_hbm.at[:, pl.ds(p0, ppw), :])

    return k(e_t_flat, idx)


@functools.partial(jax.jit, static_argnums=())
def kernel(z_e, e):
    B, C, H, W = z_e.shape
    z3 = z_e.reshape(B, C, H * W)
    idx = _tc_indices(z3, e)
    zq3 = _sc_gather(e.T.reshape(-1), idx.T, B)
    return jnp.transpose(zq3.reshape(C, H, W, B), (3, 0, 1, 2))
